# baseline (speedup 1.0000x reference)
import numpy as np
import jax
import jax.numpy as jnp
from jax import lax
from jax.experimental import pallas as pl
from jax.experimental.pallas import tpu as pltpu
from jax.experimental.pallas import tpu_sc as plsc

_NUM_GENES = 100000
_NUM_PATHWAYS = 1000
_D = 64
_NNEG = 10
_B = 16384

_NC = 2
_NS = 16
_NW = _NC * _NS
_PPW = _B // _NW
_P = 32
_NCH = _PPW // _P
_TPW = _PPW * 12


def _neg_indices():
    nkey = jax.random.key(1234)
    nk1, nk2, nk3 = jax.random.split(nkey, 3)
    neg_gg = jax.random.randint(nk1, (_B, _NNEG), 0, _NUM_GENES)
    neg_gp = jax.random.randint(nk2, (_B, _NNEG), 0, _NUM_PATHWAYS)
    neg_pg = jax.random.randint(nk3, (_B, _NNEG), 0, _NUM_GENES)
    return (neg_gg.astype(jnp.int32).reshape(-1),
            neg_gp.astype(jnp.int32).reshape(-1),
            neg_pg.astype(jnp.int32).reshape(-1))


def _sc_body(ge, pe, gw, pw, idx0, idx1, idx2, out_s,
             idx_v, hbuf, wbuf, nbuf, scores_v, spm_pe, spm_pw,
             hs0, hs1, ws0, ws1, ns0, ns1):
    sid = lax.axis_index("s")
    wid = sid * _NC + lax.axis_index("c")
    hsems = (hs0, hs1)
    wsems = (ws0, ws1)
    nsems = (ns0, ns1)

    @pl.when(sid == 0)
    def _stage():
        pltpu.sync_copy(pe, spm_pe)
        pltpu.sync_copy(pw, spm_pw)

    plsc.subcore_barrier()

    terms = (
        (ge, gw, idx0),
        (ge, spm_pw, idx1),
        (spm_pe, gw, idx2),
    )
    for t in range(3):
        pltpu.sync_copy(terms[t][2].at[pl.ds(wid * _TPW, _TPW)],
                        idx_v.at[pl.ds(t * _TPW, _TPW)])

    def start_g(c, p):
        t, k = c // _NCH, c % _NCH
        tbl_h, tbl_w, _ = terms[t]
        base = t * _TPW
        pltpu.make_async_copy(
            tbl_h.at[idx_v.at[pl.ds(base + k * _P, _P)]],
            hbuf.at[p], hsems[p]).start()
        pltpu.make_async_copy(
            tbl_w.at[idx_v.at[pl.ds(base + _PPW + k * _P, _P)]],
            wbuf.at[p], wsems[p]).start()
        pltpu.make_async_copy(
            tbl_w.at[idx_v.at[pl.ds(base + 2 * _PPW + k * _P * _NNEG,
                                    _P * _NNEG)]],
            nbuf.at[p], nsems[p]).start()

    def wait_g(c, p):
        t, _ = c // _NCH, c % _NCH
        tbl_h, tbl_w, _ = terms[t]
        pltpu.make_async_copy(tbl_h.at[idx_v.at[pl.ds(0, _P)]],
                              hbuf.at[p], hsems[p]).wait()
        pltpu.make_async_copy(tbl_w.at[idx_v.at[pl.ds(0, _P)]],
                              wbuf.at[p], wsems[p]).wait()
        pltpu.make_async_copy(tbl_w.at[idx_v.at[pl.ds(0, _P * _NNEG)]],
                              nbuf.at[p], nsems[p]).wait()

    iot = lax.broadcasted_iota(jnp.int32, (16,), 0)
    zero = jnp.zeros((16,), jnp.float32)

    def compute(c, p):
        _, k = c // _NCH, c % _NCH
        for g in range(_P // 16):
            rows = iot + g * 16
            rows_n = rows * _NNEG

            def dbody(d, accs):
                dcol = iot * 0 + d
                hv = plsc.load_gather(hbuf.at[p], [rows, dcol])
                wv = plsc.load_gather(wbuf.at[p], [rows, dcol])
                new = [accs[0] + hv * wv]
                for n in range(_NNEG):
                    nv = plsc.load_gather(nbuf.at[p], [rows_n + n, dcol])
                    new.append(accs[n + 1] + hv * nv)
                return tuple(new)

            accs = lax.fori_loop(0, _D, dbody, (zero,) * (_NNEG + 1))
            pb = k * _P + g * 16
            for j in range(_NNEG + 1):
                scores_v[j, pl.ds(pb, 16)] = accs[j]

    n = 3 * _NCH
    start_g(0, 0)
    start_g(1, 1)
    for c in range(n):
        p = c & 1
        wait_g(c, p)
        compute(c, p)
        if c + 2 < n:
            start_g(c + 2, p)
        if c % _NCH == _NCH - 1:
            pltpu.sync_copy(scores_v, out_s.at[c // _NCH, wid])


def _sc_scores(ge, pe, gw, pw, idx0, idx1, idx2):
    mesh = plsc.VectorSubcoreMesh(core_axis_name="c", subcore_axis_name="s")
    return pl.kernel(
        _sc_body,
        mesh=mesh,
        compiler_params=pltpu.CompilerParams(use_tc_tiling_on_sc=False,
                                             needs_layout_passes=False),
        out_type=jax.ShapeDtypeStruct((3, _NW, 16, _PPW), jnp.float32),
        scratch_types=(
            [pltpu.VMEM((3 * _TPW,), jnp.int32),
             pltpu.VMEM((2, _P, _D), jnp.float32),
             pltpu.VMEM((2, _P, _D), jnp.float32),
             pltpu.VMEM((2, _P * _NNEG, _D), jnp.float32),
             pltpu.VMEM((16, _PPW), jnp.float32),
             pltpu.VMEM_SHARED((_NUM_PATHWAYS, _D), jnp.float32),
             pltpu.VMEM_SHARED((_NUM_PATHWAYS, _D), jnp.float32)]
            + [pltpu.SemaphoreType.DMA] * 6),
    )(ge, pe, gw, pw, idx0, idx1, idx2)


def _loss_body(s_ref, out_ref):
    x = s_ref[:]
    row = lax.broadcasted_iota(jnp.int32, x.shape, 0)
    j = lax.rem(row, 16)
    valid = j < _NNEG + 1
    xs = jnp.where(valid, x, 0.0)
    v = jax.nn.log_sigmoid(jnp.where(j > 0, -xs, xs))
    wt = jnp.where(row < 2 * _NW * 16, 1.0, 0.5)
    contrib = jnp.where(valid, v * wt, 0.0)
    out_ref[0, 0] = -jnp.sum(contrib) / _B


def _loss_from_scores(s2):
    return pl.pallas_call(
        _loss_body,
        out_specs=pl.BlockSpec(memory_space=pltpu.SMEM),
        out_shape=jax.ShapeDtypeStruct((1, 1), jnp.float32),
    )(s2)


def kernel(gene_embeds, pathway_embeds, gene_weights, pathway_weights,
           gene_gene_pairs, gene_pathway_pairs, pathway_gene_pairs):
    i32 = jnp.int32
    src = gene_gene_pairs[0].astype(i32)
    ctx = gene_gene_pairs[1].astype(i32)
    g = gene_pathway_pairs[0].astype(i32)
    p = gene_pathway_pairs[1].astype(i32)
    p2 = pathway_gene_pairs[0].astype(i32)
    g2 = pathway_gene_pairs[1].astype(i32)

    neg_gg, neg_gp, neg_pg = _neg_indices()

    def lay(h, w, ng):
        h = h.reshape(_NW, _PPW)
        w = w.reshape(_NW, _PPW)
        ng = ng.reshape(_NW, _PPW * _NNEG)
        return jnp.concatenate([h, w, ng], axis=1).reshape(-1)

    idx0 = lay(src, ctx, neg_gg)
    idx1 = lay(g, p, neg_gp)
    idx2 = lay(p2, g2, neg_pg)

    scores = _sc_scores(gene_embeds, pathway_embeds, gene_weights,
                        pathway_weights, idx0, idx1, idx2)
    return _loss_from_scores(scores.reshape(3 * _NW * 16, _PPW))[0, 0]


# speedup vs baseline: 1.1168x; 1.1168x over previous
import numpy as np
import jax
import jax.numpy as jnp
from jax import lax
from jax.experimental import pallas as pl
from jax.experimental.pallas import tpu as pltpu
from jax.experimental.pallas import tpu_sc as plsc

_NUM_GENES = 100000
_NUM_PATHWAYS = 1000
_D = 64
_NNEG = 10
_B = 16384

_NC = 2
_NS = 16
_NW = _NC * _NS
_PPW = _B // _NW
_P = 32
_NCH = _PPW // _P
_TPW = _PPW * 12


def _neg_indices():
    nkey = jax.random.key(1234)
    nk1, nk2, nk3 = jax.random.split(nkey, 3)
    neg_gg = jax.random.randint(nk1, (_B, _NNEG), 0, _NUM_GENES)
    neg_gp = jax.random.randint(nk2, (_B, _NNEG), 0, _NUM_PATHWAYS)
    neg_pg = jax.random.randint(nk3, (_B, _NNEG), 0, _NUM_GENES)
    return (neg_gg.astype(jnp.int32).reshape(-1),
            neg_gp.astype(jnp.int32).reshape(-1),
            neg_pg.astype(jnp.int32).reshape(-1))


def _sc_body(ge, pe, gw, pw, idx0, idx1, idx2, out_s,
             idx_v, hbuf, wbuf, nbuf, scores_v, spm_pw,
             hs0, hs1, hs2, ws0, ws1, ws2, ns0, ns1, ns2):
    sid = lax.axis_index("s")
    wid = sid * _NC + lax.axis_index("c")
    hsems = (hs0, hs1, hs2)
    wsems = (ws0, ws1, ws2)
    nsems = (ns0, ns1, ns2)

    @pl.when(sid == 0)
    def _stage():
        pltpu.sync_copy(pw, spm_pw)

    plsc.subcore_barrier()

    terms = (
        (ge, gw, idx0),
        (ge, spm_pw, idx1),
        (pe, gw, idx2),
    )
    for t in range(3):
        pltpu.sync_copy(terms[t][2].at[pl.ds(wid * _TPW, _TPW)],
                        idx_v.at[pl.ds(t * _TPW, _TPW)])

    def start_g(c, p):
        t, k = c // _NCH, c % _NCH
        tbl_h, tbl_w, _ = terms[t]
        base = t * _TPW
        pltpu.make_async_copy(
            tbl_h.at[idx_v.at[pl.ds(base + k * _P, _P)]],
            hbuf.at[p], hsems[p]).start()
        pltpu.make_async_copy(
            tbl_w.at[idx_v.at[pl.ds(base + _PPW + k * _P, _P)]],
            wbuf.at[p], wsems[p]).start()
        pltpu.make_async_copy(
            tbl_w.at[idx_v.at[pl.ds(base + 2 * _PPW + k * _P * _NNEG,
                                    _P * _NNEG)]],
            nbuf.at[p], nsems[p]).start()

    def wait_g(c, p):
        t, _ = c // _NCH, c % _NCH
        tbl_h, tbl_w, _ = terms[t]
        pltpu.make_async_copy(tbl_h.at[idx_v.at[pl.ds(0, _P)]],
                              hbuf.at[p], hsems[p]).wait()
        pltpu.make_async_copy(tbl_w.at[idx_v.at[pl.ds(0, _P)]],
                              wbuf.at[p], wsems[p]).wait()
        pltpu.make_async_copy(tbl_w.at[idx_v.at[pl.ds(0, _P * _NNEG)]],
                              nbuf.at[p], nsems[p]).wait()

    iot = lax.broadcasted_iota(jnp.int32, (16,), 0)
    zero = jnp.zeros((16,), jnp.float32)

    def compute(c, p):
        _, k = c // _NCH, c % _NCH

        def gbody(g, _):
            rows = iot + g * 16
            rows_n = rows * _NNEG

            def dbody(q, accs):
                accs = list(accs)
                for dd in range(2):
                    dcol = iot * 0 + (q * 2 + dd)
                    hv = plsc.load_gather(hbuf.at[p], [rows, dcol])
                    wv = plsc.load_gather(wbuf.at[p], [rows, dcol])
                    accs[0] = accs[0] + hv * wv
                    for n in range(_NNEG):
                        nv = plsc.load_gather(nbuf.at[p], [rows_n + n, dcol])
                        accs[n + 1] = accs[n + 1] + hv * nv
                return tuple(accs)

            accs = lax.fori_loop(0, _D // 2, dbody, (zero,) * (_NNEG + 1))
            pb = k * _P + g * 16
            for j in range(_NNEG + 1):
                scores_v[j, pl.ds(pb, 16)] = accs[j]
            return 0

        lax.fori_loop(0, _P // 16, gbody, 0)

    n = 3 * _NCH
    start_g(0, 0)
    start_g(1, 1)
    for c in range(n):
        p = c % 3
        wait_g(c, p)
        if c + 2 < n:
            start_g(c + 2, (c + 2) % 3)
        compute(c, p)
        if c % _NCH == _NCH - 1:
            pltpu.sync_copy(scores_v, out_s.at[c // _NCH, wid])


def _sc_scores(ge, pe, gw, pw, idx0, idx1, idx2):
    mesh = plsc.VectorSubcoreMesh(core_axis_name="c", subcore_axis_name="s")
    return pl.kernel(
        _sc_body,
        mesh=mesh,
        compiler_params=pltpu.CompilerParams(use_tc_tiling_on_sc=False,
                                             needs_layout_passes=False),
        out_type=jax.ShapeDtypeStruct((3, _NW, 16, _PPW), jnp.float32),
        scratch_types=(
            [pltpu.VMEM((3 * _TPW,), jnp.int32),
             pltpu.VMEM((3, _P, _D), jnp.float32),
             pltpu.VMEM((3, _P, _D), jnp.float32),
             pltpu.VMEM((3, _P * _NNEG, _D), jnp.float32),
             pltpu.VMEM((16, _PPW), jnp.float32),
             pltpu.VMEM_SHARED((_NUM_PATHWAYS, _D), jnp.float32)]
            + [pltpu.SemaphoreType.DMA] * 9),
    )(ge, pe, gw, pw, idx0, idx1, idx2)


def _loss_body(s_ref, out_ref):
    x = s_ref[:]
    row = lax.broadcasted_iota(jnp.int32, x.shape, 0)
    j = lax.rem(row, 16)
    valid = j < _NNEG + 1
    xs = jnp.where(valid, x, 0.0)
    v = jax.nn.log_sigmoid(jnp.where(j > 0, -xs, xs))
    wt = jnp.where(row < 2 * _NW * 16, 1.0, 0.5)
    contrib = jnp.where(valid, v * wt, 0.0)
    out_ref[0, 0] = -jnp.sum(contrib) / _B


def _loss_from_scores(s2):
    return pl.pallas_call(
        _loss_body,
        out_specs=pl.BlockSpec(memory_space=pltpu.SMEM),
        out_shape=jax.ShapeDtypeStruct((1, 1), jnp.float32),
    )(s2)


def kernel(gene_embeds, pathway_embeds, gene_weights, pathway_weights,
           gene_gene_pairs, gene_pathway_pairs, pathway_gene_pairs):
    i32 = jnp.int32
    src = gene_gene_pairs[0].astype(i32)
    ctx = gene_gene_pairs[1].astype(i32)
    g = gene_pathway_pairs[0].astype(i32)
    p = gene_pathway_pairs[1].astype(i32)
    p2 = pathway_gene_pairs[0].astype(i32)
    g2 = pathway_gene_pairs[1].astype(i32)

    neg_gg, neg_gp, neg_pg = _neg_indices()

    def lay(h, w, ng):
        h = h.reshape(_NW, _PPW)
        w = w.reshape(_NW, _PPW)
        ng = ng.reshape(_NW, _PPW * _NNEG)
        return jnp.concatenate([h, w, ng], axis=1).reshape(-1)

    idx0 = lay(src, ctx, neg_gg)
    idx1 = lay(g, p, neg_gp)
    idx2 = lay(p2, g2, neg_pg)

    scores = _sc_scores(gene_embeds, pathway_embeds, gene_weights,
                        pathway_weights, idx0, idx1, idx2)
    return _loss_from_scores(scores.reshape(3 * _NW * 16, _PPW))[0, 0]


# speedup vs baseline: 1.1771x; 1.0540x over previous
import numpy as np
import jax
import jax.numpy as jnp
from jax import lax
from jax.experimental import pallas as pl
from jax.experimental.pallas import tpu as pltpu
from jax.experimental.pallas import tpu_sc as plsc

_NUM_GENES = 100000
_NUM_PATHWAYS = 1000
_D = 64
_NNEG = 10
_B = 16384

_NC = 2
_NS = 16
_NW = _NC * _NS
_CH = 512


def _neg_indices():
    nkey = jax.random.key(1234)
    nk1, nk2, nk3 = jax.random.split(nkey, 3)
    neg_gg = jax.random.randint(nk1, (_B, _NNEG), 0, _NUM_GENES)
    neg_gp = jax.random.randint(nk2, (_B, _NNEG), 0, _NUM_PATHWAYS)
    neg_pg = jax.random.randint(nk3, (_B, _NNEG), 0, _NUM_GENES)
    return (neg_gg.astype(jnp.int32).T.reshape(-1),
            neg_gp.astype(jnp.int32).T.reshape(-1),
            neg_pg.astype(jnp.int32).T.reshape(-1))


def _sc_gather_body(ge, pe, gw, pw, idx_ge, idx_pe, idx_gw, idx_pw,
                    out_h, out_w, out_n, idx_v, rows_v, spm_pe, spm_pw,
                    gsem0, gsem1, wsem0, wsem1):
    sid = lax.axis_index("s")
    wid = sid * _NC + lax.axis_index("c")
    gsems = (gsem0, gsem1)
    wsems = (wsem0, wsem1)

    @pl.when(sid == 0)
    def _stage():
        pltpu.sync_copy(pe, spm_pe)
        pltpu.sync_copy(pw, spm_pw)

    plsc.subcore_barrier()

    jobs = (
        (ge, idx_ge, 0, out_h, 0, 2 * _B),
        (spm_pe, idx_pe, 0, out_h, 2 * _B, _B),
        (gw, idx_gw, 0, out_w, 0, 2 * _B),
        (spm_pw, idx_pw, 0, out_w, 2 * _B, _B),
        (gw, idx_gw, 2 * _B, out_n, 0, 20 * _B),
        (spm_pw, idx_pw, _B, out_n, 20 * _B, 10 * _B),
    )
    chunks = []
    ibase = 0
    for tbl, idx_hbm, ioff, out_hbm, ooff, total in jobs:
        share = total // _NW
        pltpu.sync_copy(idx_hbm.at[pl.ds(ioff + wid * share, share)],
                        idx_v.at[pl.ds(ibase, share)])
        obase = ooff + wid * share
        for k in range(share // _CH):
            chunks.append((tbl, ibase + k * _CH, out_hbm, obase + k * _CH))
        ibase += share
    n = len(chunks)

    def start_g(c, p):
        tbl, io, _, _ = chunks[c]
        pltpu.make_async_copy(tbl.at[idx_v.at[pl.ds(io, _CH)]],
                              rows_v.at[p], gsems[p]).start()

    def wait_g(c, p):
        tbl, io, _, _ = chunks[c]
        pltpu.make_async_copy(tbl.at[idx_v.at[pl.ds(io, _CH)]],
                              rows_v.at[p], gsems[p]).wait()

    def start_wb(c, p):
        _, _, out_hbm, oo = chunks[c]
        pltpu.make_async_copy(rows_v.at[p], out_hbm.at[pl.ds(oo, _CH)],
                              wsems[p]).start()

    def wait_wb(c, p):
        _, _, out_hbm, oo = chunks[c]
        pltpu.make_async_copy(rows_v.at[p], out_hbm.at[pl.ds(oo, _CH)],
                              wsems[p]).wait()

    start_g(0, 0)
    for c in range(n):
        p = c & 1
        if c + 1 < n:
            if c >= 1:
                wait_wb(c - 1, (c + 1) & 1)
            start_g(c + 1, (c + 1) & 1)
        wait_g(c, p)
        start_wb(c, p)
    wait_wb(n - 2, 0 if (n - 2) % 2 == 0 else 1)
    wait_wb(n - 1, 0 if (n - 1) % 2 == 0 else 1)


def _sc_gather(ge, pe, gw, pw, idx_ge, idx_pe, idx_gw, idx_pw):
    mesh = plsc.VectorSubcoreMesh(core_axis_name="c", subcore_axis_name="s")
    return pl.kernel(
        _sc_gather_body,
        mesh=mesh,
        compiler_params=pltpu.CompilerParams(use_tc_tiling_on_sc=False),
        out_type=[
            jax.ShapeDtypeStruct((3 * _B, _D), jnp.float32),
            jax.ShapeDtypeStruct((3 * _B, _D), jnp.float32),
            jax.ShapeDtypeStruct((3 * _NNEG * _B, _D), jnp.float32),
        ],
        scratch_types=(
            [pltpu.VMEM((36 * _B // _NW,), jnp.int32),
             pltpu.VMEM((2, _CH, _D), jnp.float32),
             pltpu.VMEM_SHARED((_NUM_PATHWAYS, _D), jnp.float32),
             pltpu.VMEM_SHARED((_NUM_PATHWAYS, _D), jnp.float32)]
            + [pltpu.SemaphoreType.DMA] * 4),
    )(ge, pe, gw, pw, idx_ge, idx_pe, idx_gw, idx_pw)


_WMAP = (0, 2, 1)
_TERM_WEIGHT = (1.0, 1.0, 0.5)
_BB = 1024


def _loss_body(h_ref, w_ref, n_ref, out_ref, acc_ref):
    i = pl.program_id(0)

    @pl.when(i == 0)
    def _init():
        acc_ref[0] = 0.0

    tot = 0.0
    for t in range(3):
        h = h_ref[t]
        w = w_ref[_WMAP[t]]
        wt = _TERM_WEIGHT[t]
        pos = jnp.sum(h * w, axis=1)
        tot += wt * jnp.sum(jax.nn.log_sigmoid(pos))
        for n in range(_NNEG):
            nw = n_ref[_WMAP[t], n]
            sc = jnp.sum(h * nw, axis=1)
            tot += wt * jnp.sum(jax.nn.log_sigmoid(-sc))
    acc_ref[0] += tot

    @pl.when(i == pl.num_programs(0) - 1)
    def _fin():
        out_ref[0, 0] = -acc_ref[0] / _B


def _loss_from_rows(h3, w3, n4):
    return pl.pallas_call(
        _loss_body,
        grid=(_B // _BB,),
        in_specs=[
            pl.BlockSpec((3, _BB, _D), lambda i: (0, i, 0)),
            pl.BlockSpec((3, _BB, _D), lambda i: (0, i, 0)),
            pl.BlockSpec((3, _NNEG, _BB, _D), lambda i: (0, 0, i, 0)),
        ],
        out_specs=pl.BlockSpec((1, 1), lambda i: (0, 0),
                               memory_space=pltpu.SMEM),
        out_shape=jax.ShapeDtypeStruct((1, 1), jnp.float32),
        scratch_shapes=[pltpu.SMEM((1,), jnp.float32)],
    )(h3, w3, n4)


def kernel(gene_embeds, pathway_embeds, gene_weights, pathway_weights,
           gene_gene_pairs, gene_pathway_pairs, pathway_gene_pairs):
    i32 = jnp.int32
    src = gene_gene_pairs[0].astype(i32)
    ctx = gene_gene_pairs[1].astype(i32)
    g = gene_pathway_pairs[0].astype(i32)
    p = gene_pathway_pairs[1].astype(i32)
    p2 = pathway_gene_pairs[0].astype(i32)
    g2 = pathway_gene_pairs[1].astype(i32)

    neg_gg_t, neg_gp_t, neg_pg_t = _neg_indices()
    idx_ge = jnp.concatenate([src, g])
    idx_pe = p2
    idx_gw = jnp.concatenate([ctx, g2, neg_gg_t, neg_pg_t])
    idx_pw = jnp.concatenate([p, neg_gp_t])

    out_h, out_w, out_n = _sc_gather(
        gene_embeds, pathway_embeds, gene_weights, pathway_weights,
        idx_ge, idx_pe, idx_gw, idx_pw)

    h3 = out_h.reshape(3, _B, _D)
    w3 = out_w.reshape(3, _B, _D)
    n4 = out_n.reshape(3, _NNEG, _B, _D)
    return _loss_from_rows(h3, w3, n4)[0, 0]


# speedup vs baseline: 1.6036x; 1.3623x over previous
import numpy as np
import jax
import jax.numpy as jnp
from jax import lax
from jax.experimental import pallas as pl
from jax.experimental.pallas import tpu as pltpu
from jax.experimental.pallas import tpu_sc as plsc

_NUM_GENES = 100000
_NUM_PATHWAYS = 1000
_D = 64
_NNEG = 10
_B = 16384

_NC = 2
_NS = 16
_NW = _NC * _NS
_CH = 512


def _neg_indices():
    nkey = jax.random.key(1234)
    nk1, nk2, nk3 = jax.random.split(nkey, 3)
    neg_gg = jax.random.randint(nk1, (_B, _NNEG), 0, _NUM_GENES)
    neg_gp = jax.random.randint(nk2, (_B, _NNEG), 0, _NUM_PATHWAYS)
    neg_pg = jax.random.randint(nk3, (_B, _NNEG), 0, _NUM_GENES)
    return (neg_gg.astype(jnp.int32).T.reshape(-1),
            neg_gp.astype(jnp.int32).T.reshape(-1),
            neg_pg.astype(jnp.int32).T.reshape(-1))


def _sc_gather_body(ge, pe, gw, pw, idx_ge, idx_pe, idx_gw, idx_pw,
                    out_h, out_w, out_n, idx_v, rows_v, spm_pe, spm_pw,
                    gsem0, gsem1, wsem0, wsem1):
    sid = lax.axis_index("s")
    wid = sid * _NC + lax.axis_index("c")
    gsems = (gsem0, gsem1)
    wsems = (wsem0, wsem1)

    @pl.when(sid == 0)
    def _stage():
        pltpu.sync_copy(pe, spm_pe)
        pltpu.sync_copy(pw, spm_pw)

    plsc.subcore_barrier()

    jobs = (
        (ge, idx_ge, 0, out_h, 0, 2 * _B),
        (spm_pe, idx_pe, 0, out_h, 2 * _B, _B),
        (gw, idx_gw, 0, out_w, 0, 2 * _B),
        (spm_pw, idx_pw, 0, out_w, 2 * _B, _B),
        (gw, idx_gw, 2 * _B, out_n, 0, 20 * _B),
        (spm_pw, idx_pw, _B, out_n, 20 * _B, 10 * _B),
    )
    chunks = []
    ibase = 0
    for tbl, idx_hbm, ioff, out_hbm, ooff, total in jobs:
        share = total // _NW
        pltpu.sync_copy(idx_hbm.at[pl.ds(ioff + wid * share, share)],
                        idx_v.at[pl.ds(ibase, share)])
        obase = ooff + wid * share
        for k in range(share // _CH):
            chunks.append((tbl, ibase + k * _CH, out_hbm, obase + k * _CH))
        ibase += share
    n = len(chunks)

    def start_g(c, p):
        tbl, io, _, _ = chunks[c]
        pltpu.make_async_copy(tbl.at[idx_v.at[pl.ds(io, _CH)]],
                              rows_v.at[p], gsems[p]).start()

    def wait_g(c, p):
        tbl, io, _, _ = chunks[c]
        pltpu.make_async_copy(tbl.at[idx_v.at[pl.ds(io, _CH)]],
                              rows_v.at[p], gsems[p]).wait()

    def start_wb(c, p):
        _, _, out_hbm, oo = chunks[c]
        pltpu.make_async_copy(rows_v.at[p], out_hbm.at[pl.ds(oo, _CH)],
                              wsems[p]).start()

    def wait_wb(c, p):
        _, _, out_hbm, oo = chunks[c]
        pltpu.make_async_copy(rows_v.at[p], out_hbm.at[pl.ds(oo, _CH)],
                              wsems[p]).wait()

    start_g(0, 0)
    for c in range(n):
        p = c & 1
        if c + 1 < n:
            if c >= 1:
                wait_wb(c - 1, (c + 1) & 1)
            start_g(c + 1, (c + 1) & 1)
        wait_g(c, p)
        start_wb(c, p)
    wait_wb(n - 2, 0 if (n - 2) % 2 == 0 else 1)
    wait_wb(n - 1, 0 if (n - 1) % 2 == 0 else 1)


def _sc_gather(ge, pe, gw, pw, idx_ge, idx_pe, idx_gw, idx_pw):
    mesh = plsc.VectorSubcoreMesh(core_axis_name="c", subcore_axis_name="s")
    return pl.kernel(
        _sc_gather_body,
        mesh=mesh,
        compiler_params=pltpu.CompilerParams(use_tc_tiling_on_sc=False),
        out_type=[
            jax.ShapeDtypeStruct((3 * _B, _D), jnp.float32),
            jax.ShapeDtypeStruct((3 * _B, _D), jnp.float32),
            jax.ShapeDtypeStruct((3 * _NNEG * _B, _D), jnp.float32),
        ],
        scratch_types=(
            [pltpu.VMEM((36 * _B // _NW,), jnp.int32),
             pltpu.VMEM((2, _CH, _D), jnp.float32),
             pltpu.VMEM_SHARED((_NUM_PATHWAYS, _D), jnp.float32),
             pltpu.VMEM_SHARED((_NUM_PATHWAYS, _D), jnp.float32)]
            + [pltpu.SemaphoreType.DMA] * 4),
    )(ge, pe, gw, pw, idx_ge, idx_pe, idx_gw, idx_pw)


_WMAP = (0, 2, 1)
_TERM_WEIGHT = (1.0, 1.0, 0.5)
_BB = 1024


def _loss_body(h_ref, w_ref, n_ref, out_ref, acc_ref):
    i = pl.program_id(0)

    @pl.when(i == 0)
    def _init():
        acc_ref[0] = 0.0

    lane = lax.broadcasted_iota(jnp.int32, (_BB // 2, 2 * _D), 1)
    mlo = (lane < _D).astype(jnp.float32)

    def half_sum_logsig(prod, sign):
        s_all = jnp.sum(prod, axis=1)
        s_lo = jnp.sum(prod * mlo, axis=1)
        s_hi = s_all - s_lo
        return (jnp.sum(jax.nn.log_sigmoid(sign * s_lo))
                + jnp.sum(jax.nn.log_sigmoid(sign * s_hi)))

    tot = 0.0
    for t in range(3):
        h = h_ref[t]
        w = w_ref[_WMAP[t]]
        wt = _TERM_WEIGHT[t]
        tot += wt * half_sum_logsig(h * w, 1.0)
        for n in range(_NNEG):
            nw = n_ref[_WMAP[t], n]
            tot += wt * half_sum_logsig(h * nw, -1.0)
    acc_ref[0] += tot

    @pl.when(i == pl.num_programs(0) - 1)
    def _fin():
        out_ref[0, 0] = -acc_ref[0] / _B


def _loss_from_rows(h3, w3, n4):
    return pl.pallas_call(
        _loss_body,
        grid=(_B // _BB,),
        in_specs=[
            pl.BlockSpec((3, _BB // 2, 2 * _D), lambda i: (0, i, 0)),
            pl.BlockSpec((3, _BB // 2, 2 * _D), lambda i: (0, i, 0)),
            pl.BlockSpec((3, _NNEG, _BB // 2, 2 * _D), lambda i: (0, 0, i, 0)),
        ],
        out_specs=pl.BlockSpec((1, 1), lambda i: (0, 0),
                               memory_space=pltpu.SMEM),
        out_shape=jax.ShapeDtypeStruct((1, 1), jnp.float32),
        scratch_shapes=[pltpu.SMEM((1,), jnp.float32)],
    )(h3, w3, n4)


def kernel(gene_embeds, pathway_embeds, gene_weights, pathway_weights,
           gene_gene_pairs, gene_pathway_pairs, pathway_gene_pairs):
    i32 = jnp.int32
    src = gene_gene_pairs[0].astype(i32)
    ctx = gene_gene_pairs[1].astype(i32)
    g = gene_pathway_pairs[0].astype(i32)
    p = gene_pathway_pairs[1].astype(i32)
    p2 = pathway_gene_pairs[0].astype(i32)
    g2 = pathway_gene_pairs[1].astype(i32)

    neg_gg_t, neg_gp_t, neg_pg_t = _neg_indices()
    idx_ge = jnp.concatenate([src, g])
    idx_pe = p2
    idx_gw = jnp.concatenate([ctx, g2, neg_gg_t, neg_pg_t])
    idx_pw = jnp.concatenate([p, neg_gp_t])

    out_h, out_w, out_n = _sc_gather(
        gene_embeds, pathway_embeds, gene_weights, pathway_weights,
        idx_ge, idx_pe, idx_gw, idx_pw)

    h3 = out_h.reshape(3, _B // 2, 2 * _D)
    w3 = out_w.reshape(3, _B // 2, 2 * _D)
    n4 = out_n.reshape(3, _NNEG, _B // 2, 2 * _D)
    return _loss_from_rows(h3, w3, n4)[0, 0]
